# Initial kernel scaffold; baseline (speedup 1.0000x reference)
import jax
import jax.numpy as jnp
from jax.experimental import pallas as pl


def kernel(x, edge_index, pos, W, b):
    raise NotImplementedError("write your pallas kernel here")


# speedup vs baseline: 25.5515x; 25.5515x over previous
import functools

import jax
import jax.numpy as jnp
from jax import lax
from jax.experimental import pallas as pl
from jax.experimental.pallas import tpu as pltpu
from jax.experimental.pallas import tpu_sc as plsc

B, C, N, K, OUT = 1, 128, 10000, 32, 128

NW = 32
NPW = 320
NP = NW * NPW
G = 4
EG = G * K
NG = NPW // G


def _tc_tables_body(x_ref, w_ref, b_ref, u_ref, v_ref):
    xb = x_ref[...]
    w = w_ref[...]
    w1 = w[:, :C]
    w2 = w[:, C:]
    dn = (((0,), (1,)), ((), ()))
    u = lax.dot_general(xb, w1 - w2, dn, precision=lax.Precision.HIGHEST,
                        preferred_element_type=jnp.float32)
    u_ref[...] = u + b_ref[...]
    v_ref[...] = lax.dot_general(xb, w2, dn, precision=lax.Precision.HIGHEST,
                                 preferred_element_type=jnp.float32)


def _tc_tables(xf, W, b2d):
    return pl.pallas_call(
        _tc_tables_body,
        out_shape=[
            jax.ShapeDtypeStruct((N, OUT), jnp.float32),
            jax.ShapeDtypeStruct((N, OUT), jnp.float32),
        ],
    )(xf, W, b2d)


def _rsqrt_approx(d2):
    i = lax.bitcast_convert_type(d2, jnp.int32)
    y = lax.bitcast_convert_type(jnp.int32(0x5F3759DF) - (i >> 1), jnp.float32)
    for _ in range(3):
        y = y * (1.5 - 0.5 * d2 * y * y)
    return y


def _sc_body(u_hbm, v_hbm, ii_hbm, jj_hbm, px_hbm, py_hbm, pz_hbm, out_hbm,
             ii_v, jj_v, px_v, py_v, pz_v,
             u_buf0, v_buf0, u_buf1, v_buf1, supp_v, out_v0, out_v1,
             sem0, sem1, semo0, semo1):
    wid = lax.axis_index("s") * 2 + lax.axis_index("c")
    pltpu.sync_copy(ii_hbm.at[wid], ii_v)
    pltpu.sync_copy(jj_hbm.at[wid], jj_v)
    pltpu.sync_copy(px_hbm, px_v)
    pltpu.sync_copy(py_hbm, py_v)
    pltpu.sync_copy(pz_hbm, pz_v)

    def fire(g, ub, vb, sem):
        pltpu.make_async_copy(u_hbm.at[ii_v.at[g]], ub, sem).start()
        pltpu.make_async_copy(v_hbm.at[jj_v.at[g]], vb, sem).start()

    def drain(g, ub, vb, sem):
        pltpu.make_async_copy(u_hbm.at[ii_v.at[g]], ub, sem).wait()
        pltpu.make_async_copy(v_hbm.at[jj_v.at[g]], vb, sem).wait()

    def compute(g, gg, ub, vb, ob, sem, semo):
        nh = EG // 16
        ii16 = [ii_v[g, pl.ds(h * 16, 16)] for h in range(nh)]
        jj16 = [jj_v[g, pl.ds(h * 16, 16)] for h in range(nh)]
        dx = [plsc.load_gather(px_v, [jj16[h]])
              - plsc.load_gather(px_v, [ii16[h]]) for h in range(nh)]
        dy = [plsc.load_gather(py_v, [jj16[h]])
              - plsc.load_gather(py_v, [ii16[h]]) for h in range(nh)]
        dz = [plsc.load_gather(pz_v, [jj16[h]])
              - plsc.load_gather(pz_v, [ii16[h]]) for h in range(nh)]
        d2 = [dx[h] * dx[h] + dy[h] * dy[h] + dz[h] * dz[h]
              for h in range(nh)]
        d2c = [jnp.maximum(v, 1e-30) for v in d2]
        i32 = [lax.bitcast_convert_type(v, jnp.int32) for v in d2c]
        y = [lax.bitcast_convert_type(jnp.int32(0x5F3759DF) - (v >> 1),
                                      jnp.float32) for v in i32]
        for _ in range(3):
            y = [y[h] * (1.5 - 0.5 * d2c[h] * y[h] * y[h])
                 for h in range(nh)]
        d = [d2[h] * y[h] for h in range(nh)]
        e = [jnp.exp(v) for v in d]
        s = [2.0 / (1.0 + v) for v in e]
        for h in range(nh):
            supp_v[pl.ds(h * 16, 16)] = s[h]
        drain(g, ub, vb, sem)
        @pl.when(gg > 0)
        def _():
            pltpu.make_async_copy(ob, out_hbm.at[pl.ds(0, G)], semo).wait()
        for nl in range(G):
            def kstep(k, acc):
                r = nl * K + k
                sk = supp_v[pl.ds(r, 16)][0]
                return tuple(
                    jnp.maximum(acc[ci],
                                (ub[r, pl.ds(ci * 16, 16)]
                                 + vb[r, pl.ds(ci * 16, 16)]) * sk)
                    for ci in range(OUT // 16))
            acc0 = tuple(jnp.zeros((16,), jnp.float32)
                         for _ in range(OUT // 16))
            acc = lax.fori_loop(0, K, kstep, acc0)
            for ci in range(OUT // 16):
                ob[nl, pl.ds(ci * 16, 16)] = acc[ci]
        pltpu.make_async_copy(
            ob, out_hbm.at[pl.ds(wid * NPW + g * G, G)], semo).start()

    fire(0, u_buf0, v_buf0, sem0)

    def pair(gg, carry):
        g0 = 2 * gg
        fire(g0 + 1, u_buf1, v_buf1, sem1)
        compute(g0, gg, u_buf0, v_buf0, out_v0, sem0, semo0)

        @pl.when(g0 + 2 < NG)
        def _():
            fire(g0 + 2, u_buf0, v_buf0, sem0)
        compute(g0 + 1, gg, u_buf1, v_buf1, out_v1, sem1, semo1)
        return carry

    lax.fori_loop(0, NG // 2, pair, 0)
    pltpu.make_async_copy(out_v0, out_hbm.at[pl.ds(0, G)], semo0).wait()
    pltpu.make_async_copy(out_v1, out_hbm.at[pl.ds(0, G)], semo1).wait()


_sc_edge_max = functools.partial(
    pl.kernel,
    mesh=plsc.VectorSubcoreMesh(core_axis_name="c", subcore_axis_name="s"),
    out_type=jax.ShapeDtypeStruct((NP, OUT), jnp.float32),
    compiler_params=pltpu.CompilerParams(needs_layout_passes=False),
    scratch_types=[
        pltpu.VMEM((NG, EG), jnp.int32),
        pltpu.VMEM((NG, EG), jnp.int32),
        pltpu.VMEM((NP,), jnp.float32),
        pltpu.VMEM((NP,), jnp.float32),
        pltpu.VMEM((NP,), jnp.float32),
        pltpu.VMEM((EG, OUT), jnp.float32),
        pltpu.VMEM((EG, OUT), jnp.float32),
        pltpu.VMEM((EG, OUT), jnp.float32),
        pltpu.VMEM((EG, OUT), jnp.float32),
        pltpu.VMEM((EG + 16,), jnp.float32),
        pltpu.VMEM((G, OUT), jnp.float32),
        pltpu.VMEM((G, OUT), jnp.float32),
        pltpu.SemaphoreType.DMA,
        pltpu.SemaphoreType.DMA,
        pltpu.SemaphoreType.DMA,
        pltpu.SemaphoreType.DMA,
    ],
)(_sc_body)


def kernel(x, edge_index, pos, W, b):
    xf = x[0, :, :, 0]
    ei = edge_index.astype(jnp.int32)
    pad = ((0, NP - N), (0, 0))
    ii = jnp.pad(ei[1, 0], pad).reshape(NW, NG, EG)
    jj = jnp.pad(ei[0, 0], pad).reshape(NW, NG, EG)
    px = jnp.pad(pos[0, 0, :, 0], (0, NP - N))
    py = jnp.pad(pos[0, 1, :, 0], (0, NP - N))
    pz = jnp.pad(pos[0, 2, :, 0], (0, NP - N))
    U, V = _tc_tables(xf, W, b.reshape(1, OUT))
    out = _sc_edge_max(U, V, ii, jj, px, py, pz)
    max_value = out[:N].T[None, :, :, None]
    return (max_value, edge_index, pos)
